# baseline (device time: 787468 ns/iter reference)
import jax
import jax.numpy as jnp
from jax import lax
from jax.experimental import pallas as pl
from jax.experimental.pallas import tpu as pltpu

N_DEV = 32


def kernel(x, w_mat):
    m, k_loc = x.shape
    _, n = w_mat.shape
    m_per = m // N_DEV

    def body(x_ref, w_ref, out_ref, send_buf, recv_buf,
             send_sems, recv_sems, credit_sem):
        my = lax.axis_index("i")
        left = lax.rem(my + N_DEV - 1, N_DEV)
        right = lax.rem(my + 1, N_DEV)

        barrier = pltpu.get_barrier_semaphore()
        pl.semaphore_signal(barrier, inc=1, device_id=(left,),
                            device_id_type=pl.DeviceIdType.MESH)
        pl.semaphore_signal(barrier, inc=1, device_id=(right,),
                            device_id_type=pl.DeviceIdType.MESH)
        pl.semaphore_wait(barrier, 2)

        def local_partial(c):
            xs = x_ref[pl.ds(c * m_per, m_per), :]
            return jnp.dot(xs, w_ref[:, :],
                           preferred_element_type=jnp.float32)

        c0 = lax.rem(my + N_DEV - 1, N_DEV)
        send_buf[0, :, :] = local_partial(c0).astype(jnp.bfloat16)

        for h in range(N_DEV - 1):
            s = h % 2
            if h >= 2:
                pl.semaphore_wait(credit_sem, 1)
            rdma = pltpu.make_async_remote_copy(
                src_ref=send_buf.at[s],
                dst_ref=recv_buf.at[s],
                send_sem=send_sems.at[s],
                recv_sem=recv_sems.at[s],
                device_id=(right,),
                device_id_type=pl.DeviceIdType.MESH,
            )
            rdma.start()
            rdma.wait()

            c_in = lax.rem(my - h - 2 + 2 * N_DEV, N_DEV)
            acc = recv_buf[s, :, :].astype(jnp.float32) + local_partial(c_in)
            if h < N_DEV - 2:
                send_buf[(h + 1) % 2, :, :] = acc.astype(jnp.bfloat16)
            else:
                out_ref[:, :] = jnp.maximum(acc, 0.0)
            if h <= N_DEV - 4:
                pl.semaphore_signal(credit_sem, inc=1, device_id=(left,),
                                    device_id_type=pl.DeviceIdType.MESH)

    return pl.pallas_call(
        body,
        out_shape=jax.ShapeDtypeStruct((m_per, n), jnp.float32),
        in_specs=[
            pl.BlockSpec(memory_space=pltpu.VMEM),
            pl.BlockSpec(memory_space=pltpu.VMEM),
        ],
        out_specs=pl.BlockSpec(memory_space=pltpu.VMEM),
        scratch_shapes=[
            pltpu.VMEM((2, m // N_DEV, n), jnp.bfloat16),
            pltpu.VMEM((2, m // N_DEV, n), jnp.bfloat16),
            pltpu.SemaphoreType.DMA((2,)),
            pltpu.SemaphoreType.DMA((2,)),
            pltpu.SemaphoreType.REGULAR,
        ],
        compiler_params=pltpu.CompilerParams(collective_id=0),
    )(x, w_mat)


# device time: 769332 ns/iter; 1.0236x vs baseline; 1.0236x over previous
import jax
import jax.numpy as jnp
from jax import lax
from jax.experimental import pallas as pl
from jax.experimental.pallas import tpu as pltpu

N_DEV = 32


def kernel(x, w_mat):
    m, k_loc = x.shape
    _, n = w_mat.shape
    m_per = m // N_DEV
    n_half = n // 2

    def body(x_ref, w_ref, out_ref,
             send_r, recv_r, send_l, recv_l,
             send_sems_r, recv_sems_r, send_sems_l, recv_sems_l,
             credit_r, credit_l):
        my = lax.axis_index("i")
        left = lax.rem(my + N_DEV - 1, N_DEV)
        right = lax.rem(my + 1, N_DEV)

        barrier = pltpu.get_barrier_semaphore()
        pl.semaphore_signal(barrier, inc=1, device_id=(left,),
                            device_id_type=pl.DeviceIdType.MESH)
        pl.semaphore_signal(barrier, inc=1, device_id=(right,),
                            device_id_type=pl.DeviceIdType.MESH)
        pl.semaphore_wait(barrier, 2)

        def partial_r(c):
            xs = x_ref[pl.ds(c * m_per, m_per), :]
            return jnp.dot(xs, w_ref[:, :n_half],
                           preferred_element_type=jnp.float32)

        def partial_l(c):
            xs = x_ref[pl.ds(c * m_per, m_per), :]
            return jnp.dot(xs, w_ref[:, n_half:],
                           preferred_element_type=jnp.float32)

        send_r[0, :, :] = partial_r(
            lax.rem(my + N_DEV - 1, N_DEV)).astype(jnp.bfloat16)
        send_l[0, :, :] = partial_l(
            lax.rem(my + 1, N_DEV)).astype(jnp.bfloat16)

        for h in range(N_DEV - 1):
            s = h % 2
            if h >= 2:
                pl.semaphore_wait(credit_r, 1)
                pl.semaphore_wait(credit_l, 1)
            rdma_r = pltpu.make_async_remote_copy(
                src_ref=send_r.at[s], dst_ref=recv_r.at[s],
                send_sem=send_sems_r.at[s], recv_sem=recv_sems_r.at[s],
                device_id=(right,), device_id_type=pl.DeviceIdType.MESH,
            )
            rdma_l = pltpu.make_async_remote_copy(
                src_ref=send_l.at[s], dst_ref=recv_l.at[s],
                send_sem=send_sems_l.at[s], recv_sem=recv_sems_l.at[s],
                device_id=(left,), device_id_type=pl.DeviceIdType.MESH,
            )
            rdma_r.start()
            rdma_l.start()

            c_in_r = lax.rem(my - h - 2 + 2 * N_DEV, N_DEV)
            c_in_l = lax.rem(my + h + 2, N_DEV)
            p_r = partial_r(c_in_r)
            p_l = partial_l(c_in_l)

            rdma_r.wait()
            rdma_l.wait()

            acc_r = recv_r[s, :, :].astype(jnp.float32) + p_r
            acc_l = recv_l[s, :, :].astype(jnp.float32) + p_l
            if h < N_DEV - 2:
                send_r[(h + 1) % 2, :, :] = acc_r.astype(jnp.bfloat16)
                send_l[(h + 1) % 2, :, :] = acc_l.astype(jnp.bfloat16)
            else:
                out_ref[:, :n_half] = jnp.maximum(acc_r, 0.0)
                out_ref[:, n_half:] = jnp.maximum(acc_l, 0.0)
            if h <= N_DEV - 4:
                pl.semaphore_signal(credit_r, inc=1, device_id=(left,),
                                    device_id_type=pl.DeviceIdType.MESH)
                pl.semaphore_signal(credit_l, inc=1, device_id=(right,),
                                    device_id_type=pl.DeviceIdType.MESH)

    buf = lambda: pltpu.VMEM((2, m // N_DEV, n // 2), jnp.bfloat16)
    return pl.pallas_call(
        body,
        out_shape=jax.ShapeDtypeStruct((m_per, n), jnp.float32),
        in_specs=[
            pl.BlockSpec(memory_space=pltpu.VMEM),
            pl.BlockSpec(memory_space=pltpu.VMEM),
        ],
        out_specs=pl.BlockSpec(memory_space=pltpu.VMEM),
        scratch_shapes=[
            buf(), buf(), buf(), buf(),
            pltpu.SemaphoreType.DMA((2,)),
            pltpu.SemaphoreType.DMA((2,)),
            pltpu.SemaphoreType.DMA((2,)),
            pltpu.SemaphoreType.DMA((2,)),
            pltpu.SemaphoreType.REGULAR,
            pltpu.SemaphoreType.REGULAR,
        ],
        compiler_params=pltpu.CompilerParams(collective_id=0),
    )(x, w_mat)


# device time: 430734 ns/iter; 1.8282x vs baseline; 1.7861x over previous
import numpy as np

import jax
import jax.numpy as jnp
from jax import lax
from jax.experimental import pallas as pl
from jax.experimental.pallas import tpu as pltpu

N_DEV = 32


def _logical_id(x, y, z):
    row = ((0, 1), (3, 2), (4, 5), (7, 6))[y]
    return 8 * z + row[x]


def _hamiltonian_cycle():
    cyc = []
    for yi, y in enumerate(range(4)):
        zs = range(4) if yi % 2 == 0 else range(3, -1, -1)
        cyc.extend(_logical_id(0, y, z) for z in zs)
    for yi, y in enumerate(range(3, -1, -1)):
        zs = range(4) if yi % 2 == 0 else range(3, -1, -1)
        cyc.extend(_logical_id(1, y, z) for z in zs)
    assert sorted(cyc) == list(range(N_DEV))
    return np.array(cyc, dtype=np.int32)


_CYC = _hamiltonian_cycle()
_CPOS = np.argsort(_CYC).astype(np.int32)


def kernel(x, w_mat):
    m, k_loc = x.shape
    _, n = w_mat.shape
    m_per = m // N_DEV
    n_half = n // 2

    cyc = jnp.asarray(_CYC)
    q = jnp.asarray(_CPOS)[lax.axis_index("i")]
    js = jnp.arange(N_DEV, dtype=jnp.int32)
    nbrs = jnp.stack([cyc[(q + 1) % N_DEV],
                      cyc[(q - 1) % N_DEV]])
    cseq_r = cyc[(q - 1 - js) % N_DEV]
    cseq_l = cyc[(q + 1 + js) % N_DEV]

    def body(x_ref, w_ref, nbrs_ref, cr_ref, cl_ref, out_ref,
             send_r, recv_r, send_l, recv_l,
             send_sems_r, recv_sems_r, send_sems_l, recv_sems_l,
             credit_r, credit_l):
        right = nbrs_ref[0]
        left = nbrs_ref[1]

        barrier = pltpu.get_barrier_semaphore()
        pl.semaphore_signal(barrier, inc=1, device_id=(left,),
                            device_id_type=pl.DeviceIdType.MESH)
        pl.semaphore_signal(barrier, inc=1, device_id=(right,),
                            device_id_type=pl.DeviceIdType.MESH)
        pl.semaphore_wait(barrier, 2)

        def partial_r(c):
            xs = x_ref[pl.ds(c * m_per, m_per), :]
            return jnp.dot(xs, w_ref[:, :n_half],
                           preferred_element_type=jnp.float32)

        def partial_l(c):
            xs = x_ref[pl.ds(c * m_per, m_per), :]
            return jnp.dot(xs, w_ref[:, n_half:],
                           preferred_element_type=jnp.float32)

        send_r[0, :, :] = partial_r(cr_ref[0]).astype(jnp.bfloat16)
        send_l[0, :, :] = partial_l(cl_ref[0]).astype(jnp.bfloat16)

        for h in range(N_DEV - 1):
            s = h % 2
            if h >= 2:
                pl.semaphore_wait(credit_r, 1)
                pl.semaphore_wait(credit_l, 1)
            rdma_r = pltpu.make_async_remote_copy(
                src_ref=send_r.at[s], dst_ref=recv_r.at[s],
                send_sem=send_sems_r.at[s], recv_sem=recv_sems_r.at[s],
                device_id=(right,), device_id_type=pl.DeviceIdType.MESH,
            )
            rdma_l = pltpu.make_async_remote_copy(
                src_ref=send_l.at[s], dst_ref=recv_l.at[s],
                send_sem=send_sems_l.at[s], recv_sem=recv_sems_l.at[s],
                device_id=(left,), device_id_type=pl.DeviceIdType.MESH,
            )
            rdma_r.start()
            rdma_l.start()

            p_r = partial_r(cr_ref[h + 1])
            p_l = partial_l(cl_ref[h + 1])

            rdma_r.wait()
            rdma_l.wait()

            acc_r = recv_r[s, :, :].astype(jnp.float32) + p_r
            acc_l = recv_l[s, :, :].astype(jnp.float32) + p_l
            if h < N_DEV - 2:
                send_r[(h + 1) % 2, :, :] = acc_r.astype(jnp.bfloat16)
                send_l[(h + 1) % 2, :, :] = acc_l.astype(jnp.bfloat16)
            else:
                out_ref[:, :n_half] = jnp.maximum(acc_r, 0.0)
                out_ref[:, n_half:] = jnp.maximum(acc_l, 0.0)
            if h <= N_DEV - 4:
                pl.semaphore_signal(credit_r, inc=1, device_id=(left,),
                                    device_id_type=pl.DeviceIdType.MESH)
                pl.semaphore_signal(credit_l, inc=1, device_id=(right,),
                                    device_id_type=pl.DeviceIdType.MESH)

    buf = lambda: pltpu.VMEM((2, m // N_DEV, n // 2), jnp.bfloat16)
    return pl.pallas_call(
        body,
        out_shape=jax.ShapeDtypeStruct((m_per, n), jnp.float32),
        in_specs=[
            pl.BlockSpec(memory_space=pltpu.VMEM),
            pl.BlockSpec(memory_space=pltpu.VMEM),
            pl.BlockSpec(memory_space=pltpu.SMEM),
            pl.BlockSpec(memory_space=pltpu.SMEM),
            pl.BlockSpec(memory_space=pltpu.SMEM),
        ],
        out_specs=pl.BlockSpec(memory_space=pltpu.VMEM),
        scratch_shapes=[
            buf(), buf(), buf(), buf(),
            pltpu.SemaphoreType.DMA((2,)),
            pltpu.SemaphoreType.DMA((2,)),
            pltpu.SemaphoreType.DMA((2,)),
            pltpu.SemaphoreType.DMA((2,)),
            pltpu.SemaphoreType.REGULAR,
            pltpu.SemaphoreType.REGULAR,
        ],
        compiler_params=pltpu.CompilerParams(collective_id=0),
    )(x, w_mat, nbrs, cseq_r, cseq_l)


# device time: 367239 ns/iter; 2.1443x vs baseline; 1.1729x over previous
import numpy as np

import jax
import jax.numpy as jnp
from jax import lax
from jax.experimental import pallas as pl
from jax.experimental.pallas import tpu as pltpu

N_DEV = 32
ORDER = (0, 2, 1, 3)


def _logical_id(x, y, z):
    row = ((0, 1), (3, 2), (4, 5), (7, 6))[y]
    return 8 * z + row[x]


def _hamiltonian_cycle():
    cyc = []
    for yi, y in enumerate(range(4)):
        zs = range(4) if yi % 2 == 0 else range(3, -1, -1)
        cyc.extend(_logical_id(0, y, z) for z in zs)
    for yi, y in enumerate(range(3, -1, -1)):
        zs = range(4) if yi % 2 == 0 else range(3, -1, -1)
        cyc.extend(_logical_id(1, y, z) for z in zs)
    assert sorted(cyc) == list(range(N_DEV))
    return np.array(cyc, dtype=np.int32)


_CYC = _hamiltonian_cycle()
_CPOS = np.argsort(_CYC).astype(np.int32)


def kernel(x, w_mat):
    m, k_loc = x.shape
    _, n = w_mat.shape
    m_per = m // N_DEV
    n_q = n // 4

    cyc = jnp.asarray(_CYC)
    q = jnp.asarray(_CPOS)[lax.axis_index("i")]
    js = jnp.arange(N_DEV, dtype=jnp.int32)
    nbrs = jnp.stack([cyc[(q + 1) % N_DEV],
                      cyc[(q - 1) % N_DEV]])
    cseq_r = cyc[(q - 1 - js) % N_DEV]
    cseq_l = cyc[(q + 1 + js) % N_DEV]

    def body(x_ref, w_ref, nbrs_ref, cr_ref, cl_ref, out_ref, *scr):
        sbuf = scr[0:4]
        rbuf = scr[4:8]
        ssem = scr[8:12]
        rsem = scr[12:16]
        cred = scr[16:20]

        right = nbrs_ref[0]
        left = nbrs_ref[1]

        def downstream(qi):
            return right if qi < 2 else left

        def upstream(qi):
            return left if qi < 2 else right

        def seq(qi):
            return cr_ref if qi < 2 else cl_ref

        barrier = pltpu.get_barrier_semaphore()
        pl.semaphore_signal(barrier, inc=1, device_id=(left,),
                            device_id_type=pl.DeviceIdType.MESH)
        pl.semaphore_signal(barrier, inc=1, device_id=(right,),
                            device_id_type=pl.DeviceIdType.MESH)
        pl.semaphore_wait(barrier, 2)

        def partial(qi, c):
            xs = x_ref[pl.ds(c * m_per, m_per), :]
            return jnp.dot(xs, w_ref[:, qi * n_q:(qi + 1) * n_q],
                           preferred_element_type=jnp.float32)

        def desc(qi, slot):
            return pltpu.make_async_remote_copy(
                src_ref=sbuf[qi].at[slot], dst_ref=rbuf[qi].at[slot],
                send_sem=ssem[qi].at[slot], recv_sem=rsem[qi].at[slot],
                device_id=(downstream(qi),),
                device_id_type=pl.DeviceIdType.MESH,
            )

        for qi in ORDER:
            sbuf[qi][0, :, :] = partial(qi, seq(qi)[0]).astype(jnp.bfloat16)
            desc(qi, 0).start()

        for qi in ORDER:
            p = partial(qi, seq(qi)[1])
            d = desc(qi, 0)
            d.wait_recv()
            acc = rbuf[qi][0, :, :].astype(jnp.float32) + p
            sbuf[qi][1, :, :] = acc.astype(jnp.bfloat16)
            pl.semaphore_signal(cred[qi], inc=1, device_id=(upstream(qi),),
                                device_id_type=pl.DeviceIdType.MESH)
            desc(qi, 1).start()

        def hop(h, carry):
            s = lax.rem(h, 2)
            s2 = lax.rem(h + 1, 2)
            for qi in ORDER:
                c_in = seq(qi)[h + 1]
                p = partial(qi, c_in)
                desc(qi, s).wait_recv()
                acc = rbuf[qi][s, :, :].astype(jnp.float32) + p
                desc(qi, s2).wait_send()
                sbuf[qi][s2, :, :] = acc.astype(jnp.bfloat16)

                @pl.when(h <= N_DEV - 4)
                def _():
                    pl.semaphore_signal(
                        cred[qi], inc=1, device_id=(upstream(qi),),
                        device_id_type=pl.DeviceIdType.MESH)

                pl.semaphore_wait(cred[qi], 1)
                desc(qi, s2).start()
            return carry

        lax.fori_loop(1, N_DEV - 2, hop, 0)

        for qi in ORDER:
            p = partial(qi, seq(qi)[N_DEV - 1])
            d = desc(qi, 0)
            d.wait_recv()
            acc = rbuf[qi][0, :, :].astype(jnp.float32) + p
            out_ref[:, qi * n_q:(qi + 1) * n_q] = jnp.maximum(acc, 0.0)
            desc(qi, 1).wait_send()
            desc(qi, 0).wait_send()

    buf = lambda: pltpu.VMEM((2, m // N_DEV, n // 4), jnp.bfloat16)
    return pl.pallas_call(
        body,
        out_shape=jax.ShapeDtypeStruct((m_per, n), jnp.float32),
        in_specs=[
            pl.BlockSpec(memory_space=pltpu.VMEM),
            pl.BlockSpec(memory_space=pltpu.VMEM),
            pl.BlockSpec(memory_space=pltpu.SMEM),
            pl.BlockSpec(memory_space=pltpu.SMEM),
            pl.BlockSpec(memory_space=pltpu.SMEM),
        ],
        out_specs=pl.BlockSpec(memory_space=pltpu.VMEM),
        scratch_shapes=(
            [buf() for _ in range(4)]
            + [buf() for _ in range(4)]
            + [pltpu.SemaphoreType.DMA((2,)) for _ in range(4)]
            + [pltpu.SemaphoreType.DMA((2,)) for _ in range(4)]
            + [pltpu.SemaphoreType.REGULAR for _ in range(4)]
        ),
        compiler_params=pltpu.CompilerParams(collective_id=0),
    )(x, w_mat, nbrs, cseq_r, cseq_l)
